# baseline (speedup 1.0000x reference)
import jax
import jax.numpy as jnp
import numpy as np
from jax import lax
from jax.experimental import pallas as pl
from jax.experimental.pallas import tpu as pltpu
from jax.experimental.pallas import tpu_sc as plsc

_RES = 256
_NVOX = _RES * _RES * _RES
_B = 4000000
_EPS = np.float32(1e-5)
_HI = np.float32(1.0) - np.float32(1e-5)

_NW = 32
_C = 8000
_NCHUNKS = _B // _C
_NG = _C // 16
_NFULL = _NCHUNKS // _NW
_EXTRA = _NCHUNKS % _NW
_NT = _NFULL + (1 if _EXTRA else 0)


def _sc_body(x_hbm, y_hbm, z_hbm, grid_hbm, out_hbm,
             x_v0, y_v0, z_v0, x_v1, y_v1, z_v1,
             idx_v0, idx_v1, gath_v0, gath_v1,
             isem0, isem1, gsem0, gsem1, osem0, osem1):
    cid = lax.axis_index("c")
    sid = lax.axis_index("s")
    wid = sid * 2 + cid
    nch = jnp.where(wid < _EXTRA, _NFULL + 1, _NFULL)
    xyz_vs = ((x_v0, y_v0, z_v0), (x_v1, y_v1, z_v1))
    idx_vs = (idx_v0, idx_v1)
    gath_vs = (gath_v0, gath_v1)
    isems = (isem0, isem1)
    gsems = (gsem0, gsem1)
    osems = (osem0, osem1)

    bases = [(wid + t * _NW) * _C for t in range(_NT)]

    def in_copies(t, s):
        xv, yv, zv = xyz_vs[s]
        return (
            pltpu.make_async_copy(x_hbm.at[pl.ds(bases[t], _C)], xv, isems[s]),
            pltpu.make_async_copy(y_hbm.at[pl.ds(bases[t], _C)], yv, isems[s]),
            pltpu.make_async_copy(z_hbm.at[pl.ds(bases[t], _C)], zv, isems[s]),
        )

    def gath_copy(t, s):
        return pltpu.make_async_copy(grid_hbm.at[idx_vs[s]], gath_vs[s],
                                     gsems[s])

    def out_copy(t, s):
        return pltpu.make_async_copy(gath_vs[s],
                                     out_hbm.at[pl.ds(bases[t], _C)],
                                     osems[s])

    def compute_chunk(s):
        xv, yv, zv = xyz_vs[s]
        idx_v = idx_vs[s]

        def grp(g, carry2):
            pos = g * 16
            x = xv[pl.ds(pos, 16)]
            y = yv[pl.ds(pos, 16)]
            z = zv[pl.ds(pos, 16)]
            xi = (x * 256.0).astype(jnp.int32)
            yi = (y * 256.0).astype(jnp.int32)
            zi = (z * 256.0).astype(jnp.int32)
            flat = xi * 65536 + yi * 256 + zi
            inv = ((x < _EPS) | (x >= _HI) | (y < _EPS) | (y >= _HI)
                   | (z < _EPS) | (z >= _HI))
            idx_v[pl.ds(pos, 16)] = jnp.where(inv, _NVOX, flat)
            return carry2

        lax.fori_loop(0, _NG, grp, 0)

    def live(t):
        return t < nch

    @pl.when(live(0))
    def _():
        for d in in_copies(0, 0):
            d.start()

    for t in range(_NT):
        s = t & 1

        @pl.when(live(t))
        def _(t=t, s=s):
            for d in in_copies(t, s):
                d.wait()

        if t + 1 < _NT:

            @pl.when(live(t + 1))
            def _(t=t, s=s):
                for d in in_copies(t + 1, 1 - s):
                    d.start()

        @pl.when(live(t))
        def _(t=t, s=s):
            compute_chunk(s)

        if t >= 1:

            @pl.when(live(t - 1))
            def _(t=t, s=s):
                gath_copy(t - 1, 1 - s).wait()
                out_copy(t - 1, 1 - s).start()

        if t >= 2:

            @pl.when(live(t - 2))
            def _(t=t, s=s):
                out_copy(t - 2, s).wait()

        @pl.when(live(t))
        def _(t=t, s=s):
            gath_copy(t, s).start()

    tl = _NT - 1

    @pl.when(live(tl))
    def _():
        gath_copy(tl, tl & 1).wait()
        out_copy(tl, tl & 1).start()
        out_copy(tl, tl & 1).wait()

    @pl.when(live(tl - 1))
    def _():
        out_copy(tl - 1, (tl - 1) & 1).wait()


@jax.jit
def _sc_call(xs, ys, zs, grid_flat):
    mesh = plsc.VectorSubcoreMesh(core_axis_name="c", subcore_axis_name="s")
    f = pl.kernel(
        _sc_body,
        out_type=jax.ShapeDtypeStruct((_B,), jnp.bool_),
        mesh=mesh,
        scratch_types=[
            pltpu.VMEM((_C,), jnp.float32),
            pltpu.VMEM((_C,), jnp.float32),
            pltpu.VMEM((_C,), jnp.float32),
            pltpu.VMEM((_C,), jnp.float32),
            pltpu.VMEM((_C,), jnp.float32),
            pltpu.VMEM((_C,), jnp.float32),
            pltpu.VMEM((_C,), jnp.int32),
            pltpu.VMEM((_C,), jnp.int32),
            pltpu.VMEM((_C,), jnp.bool_),
            pltpu.VMEM((_C,), jnp.bool_),
            pltpu.SemaphoreType.DMA,
            pltpu.SemaphoreType.DMA,
            pltpu.SemaphoreType.DMA,
            pltpu.SemaphoreType.DMA,
            pltpu.SemaphoreType.DMA,
            pltpu.SemaphoreType.DMA,
        ],
        compiler_params=pltpu.CompilerParams(needs_layout_passes=False),
    )
    return f(xs, ys, zs, grid_flat)


def kernel(pts, grid_flat):
    return _sc_call(pts[:, 0], pts[:, 1], pts[:, 2], grid_flat)
